# baseline (speedup 1.0000x reference)
import functools

import jax
import jax.numpy as jnp
from jax.experimental import pallas as pl
from jax.experimental.pallas import tpu as pltpu


_MB = 256


def _gconv_body(n, x0_ref, x1_ref, adj_ref, wt_ref, wlt_ref, b_ref, out_ref,
                v_ref, sl_ref, pacc_ref):
    k = wt_ref.shape[0]
    nh = v_ref.shape[0]
    j = pl.program_id(0)
    i = pl.program_id(1)

    @pl.when(i == 0)
    def _build_panels():
        w = wt_ref[:].T
        wl = wlt_ref[:].T
        x0 = x0_ref[0]
        x1 = x1_ref[0]
        b = b_ref[:].reshape(1, k)
        v_ref[:, :k] = jnp.dot(
            x0, w, preferred_element_type=jnp.float32
        ).astype(jnp.bfloat16)
        v_ref[:, k:] = jnp.dot(
            x1, w, preferred_element_type=jnp.float32
        ).astype(jnp.bfloat16)
        half = pl.ds(j * nh, nh)
        sl_ref[half, :k] = jnp.dot(x0, wl, preferred_element_type=jnp.float32) + b
        sl_ref[half, k:] = jnp.dot(x1, wl, preferred_element_type=jnp.float32) + b
        pad = 2 * nh - n
        if pad > 0:
            @pl.when(j == 1)
            def _zero_tail():
                v_ref[pl.ds(nh - pad, pad), :] = jnp.zeros(
                    (pad, v_ref.shape[1]), jnp.bfloat16
                )

    acc = jnp.dot(
        adj_ref[:].astype(jnp.bfloat16),
        v_ref[:],
        preferred_element_type=jnp.float32,
    )
    rows = pl.ds(i * _MB, _MB)

    @pl.when(j == 0)
    def _park_partial():
        pacc_ref[rows, :] = acc

    @pl.when(j == 1)
    def _finish():
        out_ref[:] = jnp.maximum(acc + pacc_ref[rows, :] + sl_ref[rows, :], 0.0).T


def kernel(inputs, adj_mat, weight, loop_weight, bias):
    batch, n, f = inputs.shape
    k = weight.shape[1]
    nh = ((n + 1) // 2 + 127) // 128 * 128
    nb = pl.cdiv(n, _MB)

    packed = pl.pallas_call(
        functools.partial(_gconv_body, n),
        grid=(2, nb),
        in_specs=[
            pl.BlockSpec((1, nh, f), lambda j, i: (0, j, 0)),
            pl.BlockSpec((1, nh, f), lambda j, i: (1, j, 0)),
            pl.BlockSpec((_MB, nh), lambda j, i: (i, j)),
            pl.BlockSpec((k, f), lambda j, i: (0, 0)),
            pl.BlockSpec((k, f), lambda j, i: (0, 0)),
            pl.BlockSpec((k,), lambda j, i: (0,)),
        ],
        out_specs=pl.BlockSpec((batch * k, _MB), lambda j, i: (0, i)),
        out_shape=jax.ShapeDtypeStruct((batch * k, n), jnp.float32),
        scratch_shapes=[
            pltpu.VMEM((nh, batch * k), jnp.bfloat16),
            pltpu.VMEM((nb * _MB, batch * k), jnp.float32),
            pltpu.VMEM((nb * _MB, batch * k), jnp.float32),
        ],
    )(
        inputs,
        inputs,
        adj_mat,
        weight.T,
        loop_weight.T,
        bias,
    )
    return jnp.transpose(packed.reshape(batch, k, n), (0, 2, 1))


# speedup vs baseline: 1.1850x; 1.1850x over previous
import jax
import jax.numpy as jnp
from jax.experimental import pallas as pl
from jax.experimental.pallas import tpu as pltpu


_MB = 256


def _gconv_body(x_ref, adj_ref, wt_ref, wlt_ref, b_ref, out_ref, v_ref, sl_ref):
    k = wt_ref.shape[0]
    n = v_ref.shape[0]
    i = pl.program_id(0)

    @pl.when(i == 0)
    def _build_panels():
        w = wt_ref[:].T
        wl = wlt_ref[:].T
        x0 = x_ref[pl.ds(0, n), :]
        x1 = x_ref[pl.ds(n, n), :]
        b = b_ref[:].reshape(1, k)
        v_ref[:, :k] = jnp.dot(
            x0, w, preferred_element_type=jnp.float32
        ).astype(jnp.bfloat16)
        v_ref[:, k:] = jnp.dot(
            x1, w, preferred_element_type=jnp.float32
        ).astype(jnp.bfloat16)
        sl_ref[pl.ds(0, n), :k] = (
            jnp.dot(x0, wl, preferred_element_type=jnp.float32) + b
        )
        sl_ref[pl.ds(0, n), k:] = (
            jnp.dot(x1, wl, preferred_element_type=jnp.float32) + b
        )

    acc = jnp.dot(
        adj_ref[:].astype(jnp.bfloat16),
        v_ref[:],
        preferred_element_type=jnp.float32,
    )
    out_ref[:] = jnp.maximum(acc + sl_ref[pl.ds(i * _MB, _MB), :], 0.0).T


def kernel(inputs, adj_mat, weight, loop_weight, bias):
    batch, n, f = inputs.shape
    k = weight.shape[1]

    packed = pl.pallas_call(
        _gconv_body,
        grid=(pl.cdiv(n, _MB),),
        in_specs=[
            pl.BlockSpec((batch * n, f), lambda i: (0, 0)),
            pl.BlockSpec((_MB, n), lambda i: (i, 0)),
            pl.BlockSpec((k, f), lambda i: (0, 0)),
            pl.BlockSpec((k, f), lambda i: (0, 0)),
            pl.BlockSpec((k,), lambda i: (0,)),
        ],
        out_specs=pl.BlockSpec((batch * k, _MB), lambda i: (0, i)),
        out_shape=jax.ShapeDtypeStruct((batch * k, n), jnp.float32),
        scratch_shapes=[
            pltpu.VMEM((n, batch * k), jnp.bfloat16),
            pltpu.VMEM((pl.cdiv(n, _MB) * _MB, batch * k), jnp.float32),
        ],
    )(
        inputs.reshape(batch * n, f),
        adj_mat,
        weight.T,
        loop_weight.T,
        bias,
    )
    return jnp.transpose(packed.reshape(batch, k, n), (0, 2, 1))
